# initial kernel scaffold (unmeasured)
import jax
import jax.numpy as jnp
from jax import lax
from jax.experimental import pallas as pl
from jax.experimental.pallas import tpu as pltpu


def _flash_partial(Q, K, V):
    b, q, h, d = Q.shape
    skv = K.shape[1]
    scale = d ** -0.5

    def body(q_ref, k_ref, v_ref, o_ref, m_ref, l_ref):
        qv = q_ref[0, :, 0, :].astype(jnp.bfloat16)
        kv = k_ref[0, :, 0, :].astype(jnp.bfloat16)
        vv = v_ref[0, :, 0, :].astype(jnp.bfloat16)
        s = lax.dot_general(
            qv, kv, (((1,), (1,)), ((), ())),
            preferred_element_type=jnp.float32,
        ) * scale
        m = jnp.max(s, axis=1, keepdims=True)
        p = jnp.exp(s - m)
        lsum = jnp.sum(p, axis=1, keepdims=True)
        o = lax.dot_general(
            p.astype(jnp.bfloat16), vv, (((1,), (0,)), ((), ())),
            preferred_element_type=jnp.float32,
        )
        o_ref[0, 0, :, :] = o.astype(jnp.bfloat16)
        m_ref[0, 0, :, :] = m
        l_ref[0, 0, :, :] = lsum

    return pl.pallas_call(
        body,
        grid=(b, h),
        in_specs=[
            pl.BlockSpec((1, q, 1, d), lambda i, j: (i, 0, j, 0)),
            pl.BlockSpec((1, skv, 1, d), lambda i, j: (i, 0, j, 0)),
            pl.BlockSpec((1, skv, 1, d), lambda i, j: (i, 0, j, 0)),
        ],
        out_specs=[
            pl.BlockSpec((1, 1, q, d), lambda i, j: (i, j, 0, 0)),
            pl.BlockSpec((1, 1, q, 1), lambda i, j: (i, j, 0, 0)),
            pl.BlockSpec((1, 1, q, 1), lambda i, j: (i, j, 0, 0)),
        ],
        out_shape=[
            jax.ShapeDtypeStruct((b, h, q, d), jnp.bfloat16),
            jax.ShapeDtypeStruct((b, h, q, 1), jnp.float32),
            jax.ShapeDtypeStruct((b, h, q, 1), jnp.float32),
        ],
    )(Q, K, V)


def _combine(o_un, m, l):
    b, h, q, d = o_un.shape

    def body(o_ref, m_ref, l_ref, out_ref, o_rx, m_rx, l_rx, send_sems, recv_sems):
        my_x = lax.axis_index("x")
        my_y = lax.axis_index("y")
        my_z = lax.axis_index("z")
        peer = (my_x, 1 - my_y, my_z)

        barrier = pltpu.get_barrier_semaphore()
        pl.semaphore_signal(
            barrier, inc=1, device_id=peer,
            device_id_type=pl.DeviceIdType.MESH,
        )
        pl.semaphore_wait(barrier, 1)

        copies = []
        for i, (src, dst) in enumerate(
            ((o_ref, o_rx), (m_ref, m_rx), (l_ref, l_rx))
        ):
            c = pltpu.make_async_remote_copy(
                src_ref=src, dst_ref=dst,
                send_sem=send_sems.at[i], recv_sem=recv_sems.at[i],
                device_id=peer, device_id_type=pl.DeviceIdType.MESH,
            )
            c.start()
            copies.append(c)
        for c in copies:
            c.wait()

        m0 = m_ref[...]
        m1 = m_rx[...]
        mx = jnp.maximum(m0, m1)
        e0 = jnp.exp(m0 - mx)
        e1 = jnp.exp(m1 - mx)
        denom = e0 * l_ref[...] + e1 * l_rx[...]
        comb = (
            o_ref[...].astype(jnp.float32) * e0
            + o_rx[...].astype(jnp.float32) * e1
        ) / denom
        for hh in range(h):
            out_ref[:, :, hh, :] = comb[:, hh, :, :]

    return pl.pallas_call(
        body,
        in_specs=[pl.BlockSpec(memory_space=pltpu.VMEM)] * 3,
        out_specs=pl.BlockSpec(memory_space=pltpu.VMEM),
        out_shape=jax.ShapeDtypeStruct((b, q, h, d), jnp.float32),
        scratch_shapes=[
            pltpu.VMEM((b, h, q, d), jnp.bfloat16),
            pltpu.VMEM((b, h, q, 1), jnp.float32),
            pltpu.VMEM((b, h, q, 1), jnp.float32),
            pltpu.SemaphoreType.DMA((3,)),
            pltpu.SemaphoreType.DMA((3,)),
        ],
        compiler_params=pltpu.CompilerParams(collective_id=0),
    )(o_un, m, l)


def kernel(Q, K, V):
    o_un, m, l = _flash_partial(Q, K, V)
    return _combine(o_un, m, l)


# baseline (device time: 169970 ns/iter reference)
import jax
import jax.numpy as jnp
from jax import lax
from jax.experimental import pallas as pl
from jax.experimental.pallas import tpu as pltpu


def _flash_partial(Q, K, V):
    b, q, h, d = Q.shape
    skv = K.shape[1]
    scale = d ** -0.5

    def body(q_ref, k_ref, v_ref, o_ref, m_ref, l_ref):
        for hh in range(h):
            qv = q_ref[0, :, hh, :].astype(jnp.bfloat16)
            kv = k_ref[0, :, hh, :].astype(jnp.bfloat16)
            vv = v_ref[0, :, hh, :].astype(jnp.bfloat16)
            s = lax.dot_general(
                qv, kv, (((1,), (1,)), ((), ())),
                preferred_element_type=jnp.float32,
            ) * scale
            m = jnp.max(s, axis=1, keepdims=True)
            p = jnp.exp(s - m)
            lsum = jnp.sum(p, axis=1, keepdims=True)
            o = lax.dot_general(
                p.astype(jnp.bfloat16), vv, (((1,), (0,)), ((), ())),
                preferred_element_type=jnp.float32,
            )
            o_ref[0, hh, :, :] = o.astype(jnp.bfloat16)
            m_ref[0, hh, :, :] = m
            l_ref[0, hh, :, :] = lsum

    return pl.pallas_call(
        body,
        grid=(b,),
        in_specs=[
            pl.BlockSpec((1, q, h, d), lambda i: (i, 0, 0, 0)),
            pl.BlockSpec((1, skv, h, d), lambda i: (i, 0, 0, 0)),
            pl.BlockSpec((1, skv, h, d), lambda i: (i, 0, 0, 0)),
        ],
        out_specs=[
            pl.BlockSpec((1, h, q, d), lambda i: (i, 0, 0, 0)),
            pl.BlockSpec((1, h, q, 1), lambda i: (i, 0, 0, 0)),
            pl.BlockSpec((1, h, q, 1), lambda i: (i, 0, 0, 0)),
        ],
        out_shape=[
            jax.ShapeDtypeStruct((b, h, q, d), jnp.bfloat16),
            jax.ShapeDtypeStruct((b, h, q, 1), jnp.float32),
            jax.ShapeDtypeStruct((b, h, q, 1), jnp.float32),
        ],
        compiler_params=pltpu.CompilerParams(
            vmem_limit_bytes=64 * 1024 * 1024,
        ),
    )(Q, K, V)


def _combine(o_un, m, l):
    b, h, q, d = o_un.shape

    def body(o_ref, m_ref, l_ref, out_ref, o_rx, m_rx, l_rx, send_sems, recv_sems):
        my_x = lax.axis_index("x")
        my_y = lax.axis_index("y")
        my_z = lax.axis_index("z")
        peer = (my_x, 1 - my_y, my_z)

        barrier = pltpu.get_barrier_semaphore()
        pl.semaphore_signal(
            barrier, inc=1, device_id=peer,
            device_id_type=pl.DeviceIdType.MESH,
        )
        pl.semaphore_wait(barrier, 1)

        copies = []
        for i, (src, dst) in enumerate(
            ((o_ref, o_rx), (m_ref, m_rx), (l_ref, l_rx))
        ):
            c = pltpu.make_async_remote_copy(
                src_ref=src, dst_ref=dst,
                send_sem=send_sems.at[i], recv_sem=recv_sems.at[i],
                device_id=peer, device_id_type=pl.DeviceIdType.MESH,
            )
            c.start()
            copies.append(c)
        for c in copies:
            c.wait()

        m0 = m_ref[...]
        m1 = m_rx[...]
        mx = jnp.maximum(m0, m1)
        e0 = jnp.exp(m0 - mx)
        e1 = jnp.exp(m1 - mx)
        denom = e0 * l_ref[...] + e1 * l_rx[...]
        comb = (
            o_ref[...].astype(jnp.float32) * e0
            + o_rx[...].astype(jnp.float32) * e1
        ) / denom
        for hh in range(h):
            out_ref[:, :, hh, :] = comb[:, hh, :, :]

    return pl.pallas_call(
        body,
        in_specs=[pl.BlockSpec(memory_space=pltpu.VMEM)] * 3,
        out_specs=pl.BlockSpec(memory_space=pltpu.VMEM),
        out_shape=jax.ShapeDtypeStruct((b, q, h, d), jnp.float32),
        scratch_shapes=[
            pltpu.VMEM((b, h, q, d), jnp.bfloat16),
            pltpu.VMEM((b, h, q, 1), jnp.float32),
            pltpu.VMEM((b, h, q, 1), jnp.float32),
            pltpu.SemaphoreType.DMA((3,)),
            pltpu.SemaphoreType.DMA((3,)),
        ],
        compiler_params=pltpu.CompilerParams(collective_id=0),
    )(o_un, m, l)


def kernel(Q, K, V):
    o_un, m, l = _flash_partial(Q, K, V)
    return _combine(o_un, m, l)


# device time: 119841 ns/iter; 1.4183x vs baseline; 1.4183x over previous
import jax
import jax.numpy as jnp
from jax import lax
from jax.experimental import pallas as pl
from jax.experimental.pallas import tpu as pltpu


def _flash_partial(Q, K, V):
    b, q, h, d = Q.shape
    skv = K.shape[1]
    scale = d ** -0.5

    def body(q_ref, k_ref, v_ref, o_ref, st_ref, kst, vst, ksem, vsem):
        bb = pl.program_id(0)

        def start_slice(hh, slot):
            pltpu.make_async_copy(
                k_ref.at[bb, :, hh, :], kst.at[slot], ksem.at[slot]
            ).start()
            pltpu.make_async_copy(
                v_ref.at[bb, :, hh, :], vst.at[slot], vsem.at[slot]
            ).start()

        start_slice(0, 0)
        ms = []
        ls = []
        for hh in range(h):
            slot = hh % 2
            if hh + 1 < h:
                start_slice(hh + 1, (hh + 1) % 2)
            pltpu.make_async_copy(
                k_ref.at[bb, :, hh, :], kst.at[slot], ksem.at[slot]
            ).wait()
            pltpu.make_async_copy(
                v_ref.at[bb, :, hh, :], vst.at[slot], vsem.at[slot]
            ).wait()
            qv = q_ref[0, :, hh, :].astype(jnp.bfloat16)
            kv = kst[slot].astype(jnp.bfloat16)
            vv = vst[slot].astype(jnp.bfloat16)
            s = lax.dot_general(
                qv, kv, (((1,), (1,)), ((), ())),
                preferred_element_type=jnp.float32,
            ) * scale
            m = jnp.max(s, axis=1, keepdims=True)
            p = jnp.exp(s - m)
            lsum = jnp.sum(p, axis=1, keepdims=True)
            o = lax.dot_general(
                p.astype(jnp.bfloat16), vv, (((1,), (0,)), ((), ())),
                preferred_element_type=jnp.float32,
            )
            o_ref[0, hh, :, :] = o.astype(jnp.bfloat16)
            ms.append(m)
            ls.append(lsum)
        st_ref[0] = jnp.concatenate(ms + ls, axis=1)

    return pl.pallas_call(
        body,
        grid=(b,),
        in_specs=[
            pl.BlockSpec((1, q, h, d), lambda i: (i, 0, 0, 0)),
            pl.BlockSpec(memory_space=pl.ANY),
            pl.BlockSpec(memory_space=pl.ANY),
        ],
        out_specs=[
            pl.BlockSpec((1, h, q, d), lambda i: (i, 0, 0, 0)),
            pl.BlockSpec((1, q, 2 * h), lambda i: (i, 0, 0)),
        ],
        out_shape=[
            jax.ShapeDtypeStruct((b, h, q, d), jnp.bfloat16),
            jax.ShapeDtypeStruct((b, q, 2 * h), jnp.float32),
        ],
        scratch_shapes=[
            pltpu.VMEM((2, skv, d), jnp.float32),
            pltpu.VMEM((2, skv, d), jnp.float32),
            pltpu.SemaphoreType.DMA((2,)),
            pltpu.SemaphoreType.DMA((2,)),
        ],
        compiler_params=pltpu.CompilerParams(
            vmem_limit_bytes=64 * 1024 * 1024,
        ),
    )(Q, K, V)


def _combine(o_un, st):
    b, h, q, d = o_un.shape

    def body(o_ref, st_ref, out_ref, o_rx, st_rx, send_sems, recv_sems):
        my_x = lax.axis_index("x")
        my_y = lax.axis_index("y")
        my_z = lax.axis_index("z")
        peer = (my_x, 1 - my_y, my_z)

        barrier = pltpu.get_barrier_semaphore()
        pl.semaphore_signal(
            barrier, inc=1, device_id=peer,
            device_id_type=pl.DeviceIdType.MESH,
        )
        pl.semaphore_wait(barrier, 1)

        copies = []
        for i, (src, dst) in enumerate(((o_ref, o_rx), (st_ref, st_rx))):
            c = pltpu.make_async_remote_copy(
                src_ref=src, dst_ref=dst,
                send_sem=send_sems.at[i], recv_sem=recv_sems.at[i],
                device_id=peer, device_id_type=pl.DeviceIdType.MESH,
            )
            c.start()
            copies.append(c)
        for c in copies:
            c.wait()

        m0 = st_ref[:, :, :h]
        l0 = st_ref[:, :, h:]
        m1 = st_rx[:, :, :h]
        l1 = st_rx[:, :, h:]
        mx = jnp.maximum(m0, m1)
        e0 = jnp.exp(m0 - mx)
        e1 = jnp.exp(m1 - mx)
        denom = e0 * l0 + e1 * l1
        w0 = e0 / denom
        w1 = e1 / denom
        for hh in range(h):
            out_ref[:, :, hh, :] = (
                o_ref[:, hh].astype(jnp.float32) * w0[:, :, hh:hh + 1]
                + o_rx[:, hh].astype(jnp.float32) * w1[:, :, hh:hh + 1]
            )

    return pl.pallas_call(
        body,
        in_specs=[pl.BlockSpec(memory_space=pltpu.VMEM)] * 2,
        out_specs=pl.BlockSpec(memory_space=pltpu.VMEM),
        out_shape=jax.ShapeDtypeStruct((b, q, h, d), jnp.float32),
        scratch_shapes=[
            pltpu.VMEM((b, h, q, d), jnp.bfloat16),
            pltpu.VMEM((b, q, 2 * h), jnp.float32),
            pltpu.SemaphoreType.DMA((2,)),
            pltpu.SemaphoreType.DMA((2,)),
        ],
        compiler_params=pltpu.CompilerParams(collective_id=0),
    )(o_un, st)


def kernel(Q, K, V):
    o_un, st = _flash_partial(Q, K, V)
    return _combine(o_un, st)


# device time: 67160 ns/iter; 2.5308x vs baseline; 1.7844x over previous
import jax
import jax.numpy as jnp
from jax import lax
from jax.experimental import pallas as pl
from jax.experimental.pallas import tpu as pltpu


def _flash_partial(Q, K, V):
    b, q, h, d = Q.shape
    skv = K.shape[1]
    scale = d ** -0.5

    def body(q_ref, k_ref, v_ref, o_ref, st_ref, kst, vst, ksem, vsem):
        bb = pl.program_id(0)

        def start_slice(bidx, hh, slot):
            pltpu.make_async_copy(
                k_ref.at[bidx, :, hh, :], kst.at[slot], ksem.at[slot]
            ).start()
            pltpu.make_async_copy(
                v_ref.at[bidx, :, hh, :], vst.at[slot], vsem.at[slot]
            ).start()

        nbuf = 8

        @pl.when(bb == 0)
        def _():
            for j in range(nbuf - 1):
                start_slice(bb, j, j)

        ms = []
        ls = []
        for hh in range(h):
            slot = hh % nbuf
            t = hh + nbuf - 1
            if t < h:
                start_slice(bb, t, t % nbuf)
            else:
                @pl.when(bb + 1 < b)
                def _():
                    start_slice(bb + 1, t - h, (t - h) % nbuf)
            pltpu.make_async_copy(
                k_ref.at[bb, :, hh, :], kst.at[slot], ksem.at[slot]
            ).wait()
            pltpu.make_async_copy(
                v_ref.at[bb, :, hh, :], vst.at[slot], vsem.at[slot]
            ).wait()
            qv = q_ref[0, :, hh, :].astype(jnp.bfloat16)
            kv = kst[slot].astype(jnp.bfloat16)
            vv = vst[slot].astype(jnp.bfloat16)
            s = lax.dot_general(
                qv, kv, (((1,), (1,)), ((), ())),
                preferred_element_type=jnp.float32,
            ) * scale
            m = jnp.max(s, axis=1, keepdims=True)
            p = jnp.exp(s - m)
            lsum = jnp.sum(p, axis=1, keepdims=True)
            o = lax.dot_general(
                p.astype(jnp.bfloat16), vv, (((1,), (0,)), ((), ())),
                preferred_element_type=jnp.float32,
            )
            o_ref[0, hh, :, :] = o.astype(jnp.bfloat16)
            ms.append(m)
            ls.append(lsum)
        st_ref[0] = jnp.concatenate(ms + ls, axis=1)

    return pl.pallas_call(
        body,
        grid=(b,),
        in_specs=[
            pl.BlockSpec((1, q, h, d), lambda i: (i, 0, 0, 0)),
            pl.BlockSpec(memory_space=pl.ANY),
            pl.BlockSpec(memory_space=pl.ANY),
        ],
        out_specs=[
            pl.BlockSpec((1, h, q, d), lambda i: (i, 0, 0, 0)),
            pl.BlockSpec((1, q, 2 * h), lambda i: (i, 0, 0)),
        ],
        out_shape=[
            jax.ShapeDtypeStruct((b, h, q, d), jnp.bfloat16),
            jax.ShapeDtypeStruct((b, q, 2 * h), jnp.float32),
        ],
        scratch_shapes=[
            pltpu.VMEM((8, skv, d), jnp.float32),
            pltpu.VMEM((8, skv, d), jnp.float32),
            pltpu.SemaphoreType.DMA((8,)),
            pltpu.SemaphoreType.DMA((8,)),
        ],
        compiler_params=pltpu.CompilerParams(
            vmem_limit_bytes=64 * 1024 * 1024,
        ),
    )(Q, K, V)


def _combine(o_un, st):
    b, h, q, d = o_un.shape

    def body(o_ref, st_ref, out_ref, o_rx, st_rx, send_sems, recv_sems):
        my_x = lax.axis_index("x")
        my_y = lax.axis_index("y")
        my_z = lax.axis_index("z")
        peer = (my_x, 1 - my_y, my_z)

        barrier = pltpu.get_barrier_semaphore()
        pl.semaphore_signal(
            barrier, inc=1, device_id=peer,
            device_id_type=pl.DeviceIdType.MESH,
        )
        pl.semaphore_wait(barrier, 1)

        copies = []
        for i, (src, dst) in enumerate(((o_ref, o_rx), (st_ref, st_rx))):
            c = pltpu.make_async_remote_copy(
                src_ref=src, dst_ref=dst,
                send_sem=send_sems.at[i], recv_sem=recv_sems.at[i],
                device_id=peer, device_id_type=pl.DeviceIdType.MESH,
            )
            c.start()
            copies.append(c)
        for c in copies:
            c.wait()

        m0 = st_ref[:, :, :h]
        l0 = st_ref[:, :, h:]
        m1 = st_rx[:, :, :h]
        l1 = st_rx[:, :, h:]
        mx = jnp.maximum(m0, m1)
        e0 = jnp.exp(m0 - mx)
        e1 = jnp.exp(m1 - mx)
        denom = e0 * l0 + e1 * l1
        w0 = e0 / denom
        w1 = e1 / denom
        for hh in range(h):
            out_ref[:, :, hh, :] = (
                o_ref[:, hh].astype(jnp.float32) * w0[:, :, hh:hh + 1]
                + o_rx[:, hh].astype(jnp.float32) * w1[:, :, hh:hh + 1]
            )

    return pl.pallas_call(
        body,
        in_specs=[pl.BlockSpec(memory_space=pltpu.VMEM)] * 2,
        out_specs=pl.BlockSpec(memory_space=pltpu.VMEM),
        out_shape=jax.ShapeDtypeStruct((b, q, h, d), jnp.float32),
        scratch_shapes=[
            pltpu.VMEM((b, h, q, d), jnp.bfloat16),
            pltpu.VMEM((b, q, 2 * h), jnp.float32),
            pltpu.SemaphoreType.DMA((2,)),
            pltpu.SemaphoreType.DMA((2,)),
        ],
        compiler_params=pltpu.CompilerParams(collective_id=0),
    )(o_un, st)


def kernel(Q, K, V):
    o_un, st = _flash_partial(Q, K, V)
    return _combine(o_un, st)


# device time: 62915 ns/iter; 2.7016x vs baseline; 1.0675x over previous
import jax
import jax.numpy as jnp
from jax import lax
from jax.experimental import pallas as pl
from jax.experimental.pallas import tpu as pltpu


def kernel(Q, K, V):
    b, q, h, d = Q.shape
    skv = K.shape[1]
    scale = d ** -0.5

    def body(q_ref, k_ref, v_ref, out_ref,
             kst, vst, ksem, vsem,
             o_loc, st_loc, o_rx, st_rx,
             osend, orecv, ssend, srecv):
        bb = pl.program_id(0)
        my_x = lax.axis_index("x")
        my_y = lax.axis_index("y")
        my_z = lax.axis_index("z")
        peer = (my_x, 1 - my_y, my_z)

        def rdma_o(i):
            return pltpu.make_async_remote_copy(
                src_ref=o_loc.at[i], dst_ref=o_rx.at[i],
                send_sem=osend.at[i], recv_sem=orecv.at[i],
                device_id=peer, device_id_type=pl.DeviceIdType.MESH,
            )

        def rdma_st(i):
            return pltpu.make_async_remote_copy(
                src_ref=st_loc.at[i], dst_ref=st_rx.at[i],
                send_sem=ssend.at[i], recv_sem=srecv.at[i],
                device_id=peer, device_id_type=pl.DeviceIdType.MESH,
            )

        @pl.when(bb == 0)
        def _():
            barrier = pltpu.get_barrier_semaphore()
            pl.semaphore_signal(
                barrier, inc=1, device_id=peer,
                device_id_type=pl.DeviceIdType.MESH,
            )
            pl.semaphore_wait(barrier, 1)

        def start_slice(bidx, hh, slot):
            pltpu.make_async_copy(
                k_ref.at[bidx, :, hh, :], kst.at[slot], ksem.at[slot]
            ).start()
            pltpu.make_async_copy(
                v_ref.at[bidx, :, hh, :], vst.at[slot], vsem.at[slot]
            ).start()

        nbuf = 8

        @pl.when(bb == 0)
        def _():
            for j in range(nbuf - 1):
                start_slice(bb, j, j)

        ms = []
        ls = []
        for hh in range(h):
            slot = hh % nbuf
            t = hh + nbuf - 1
            if t < h:
                start_slice(bb, t, t % nbuf)
            else:
                @pl.when(bb + 1 < b)
                def _():
                    start_slice(bb + 1, t - h, (t - h) % nbuf)
            pltpu.make_async_copy(
                k_ref.at[bb, :, hh, :], kst.at[slot], ksem.at[slot]
            ).wait()
            pltpu.make_async_copy(
                v_ref.at[bb, :, hh, :], vst.at[slot], vsem.at[slot]
            ).wait()
            qv = q_ref[0, :, hh, :].astype(jnp.bfloat16)
            kv = kst[slot].astype(jnp.bfloat16)
            vv = vst[slot].astype(jnp.bfloat16)
            s = lax.dot_general(
                qv, kv, (((1,), (1,)), ((), ())),
                preferred_element_type=jnp.float32,
            ) * scale
            m = jnp.max(s, axis=1, keepdims=True)
            p = jnp.exp(s - m)
            lsum = jnp.sum(p, axis=1, keepdims=True)
            o = lax.dot_general(
                p.astype(jnp.bfloat16), vv, (((1,), (0,)), ((), ())),
                preferred_element_type=jnp.float32,
            )
            o_loc[bb, hh, :, :] = o.astype(jnp.bfloat16)
            ms.append(m)
            ls.append(lsum)
        st_loc[bb] = jnp.concatenate(ms + ls, axis=1)

        rdma_o(bb).start()
        rdma_st(bb).start()

        @pl.when(bb == b - 1)
        def _():
            for i in range(b):
                rdma_o(i).wait()
                rdma_st(i).wait()
            m0 = st_loc[:, :, :h]
            l0 = st_loc[:, :, h:]
            m1 = st_rx[:, :, :h]
            l1 = st_rx[:, :, h:]
            mx = jnp.maximum(m0, m1)
            e0 = jnp.exp(m0 - mx)
            e1 = jnp.exp(m1 - mx)
            denom = e0 * l0 + e1 * l1
            w0 = e0 / denom
            w1 = e1 / denom
            for hh in range(h):
                out_ref[:, :, hh, :] = (
                    o_loc[:, hh].astype(jnp.float32) * w0[:, :, hh:hh + 1]
                    + o_rx[:, hh].astype(jnp.float32) * w1[:, :, hh:hh + 1]
                )

    return pl.pallas_call(
        body,
        grid=(b,),
        in_specs=[
            pl.BlockSpec((1, q, h, d), lambda i: (i, 0, 0, 0)),
            pl.BlockSpec(memory_space=pl.ANY),
            pl.BlockSpec(memory_space=pl.ANY),
        ],
        out_specs=pl.BlockSpec((b, q, h, d), lambda i: (0, 0, 0, 0)),
        out_shape=jax.ShapeDtypeStruct((b, q, h, d), jnp.float32),
        scratch_shapes=[
            pltpu.VMEM((8, skv, d), jnp.float32),
            pltpu.VMEM((8, skv, d), jnp.float32),
            pltpu.SemaphoreType.DMA((8,)),
            pltpu.SemaphoreType.DMA((8,)),
            pltpu.VMEM((b, h, q, d), jnp.bfloat16),
            pltpu.VMEM((b, q, 2 * h), jnp.float32),
            pltpu.VMEM((b, h, q, d), jnp.bfloat16),
            pltpu.VMEM((b, q, 2 * h), jnp.float32),
            pltpu.SemaphoreType.DMA((b,)),
            pltpu.SemaphoreType.DMA((b,)),
            pltpu.SemaphoreType.DMA((b,)),
            pltpu.SemaphoreType.DMA((b,)),
        ],
        compiler_params=pltpu.CompilerParams(
            collective_id=0,
            vmem_limit_bytes=64 * 1024 * 1024,
        ),
    )(Q, K, V)


# device time: 53310 ns/iter; 3.1883x vs baseline; 1.1802x over previous
import jax
import jax.numpy as jnp
from jax import lax
from jax.experimental import pallas as pl
from jax.experimental.pallas import tpu as pltpu


def kernel(Q, K, V):
    b, q, h, d = Q.shape
    skv = K.shape[1]
    scale = d ** -0.5

    gh = 4
    ng = h // gh

    def body(q_ref, k_ref, v_ref, out_ref,
             kst, vst, ksem, vsem,
             o_loc, st_loc, o_rx, st_rx,
             osend, orecv, ssend, srecv):
        bb = pl.program_id(0)
        my_x = lax.axis_index("x")
        my_y = lax.axis_index("y")
        my_z = lax.axis_index("z")
        peer = (my_x, 1 - my_y, my_z)

        def rdma_o(i):
            return pltpu.make_async_remote_copy(
                src_ref=o_loc.at[i], dst_ref=o_rx.at[i],
                send_sem=osend.at[i], recv_sem=orecv.at[i],
                device_id=peer, device_id_type=pl.DeviceIdType.MESH,
            )

        def rdma_st(i):
            return pltpu.make_async_remote_copy(
                src_ref=st_loc.at[i], dst_ref=st_rx.at[i],
                send_sem=ssend.at[i], recv_sem=srecv.at[i],
                device_id=peer, device_id_type=pl.DeviceIdType.MESH,
            )

        @pl.when(bb == 0)
        def _():
            barrier = pltpu.get_barrier_semaphore()
            pl.semaphore_signal(
                barrier, inc=1, device_id=peer,
                device_id_type=pl.DeviceIdType.MESH,
            )
            pl.semaphore_wait(barrier, 1)

        k2 = k_ref.reshape(b, skv, ng, gh * d)
        v2 = v_ref.reshape(b, skv, ng, gh * d)

        def start_slice(bidx, g, slot):
            pltpu.make_async_copy(
                k2.at[bidx, :, g, :], kst.at[slot], ksem.at[slot]
            ).start()
            pltpu.make_async_copy(
                v2.at[bidx, :, g, :], vst.at[slot], vsem.at[slot]
            ).start()

        nbuf = 4

        @pl.when(bb == 0)
        def _():
            for j in range(nbuf - 1):
                start_slice(bb, j, j)

        ms = []
        ls = []
        for g in range(ng):
            slot = g % nbuf
            t = g + nbuf - 1
            if t < ng:
                start_slice(bb, t, t % nbuf)
            else:
                @pl.when(bb + 1 < b)
                def _():
                    start_slice(bb + 1, t - ng, (t - ng) % nbuf)
            pltpu.make_async_copy(
                k2.at[bb, :, g, :], kst.at[slot], ksem.at[slot]
            ).wait()
            pltpu.make_async_copy(
                v2.at[bb, :, g, :], vst.at[slot], vsem.at[slot]
            ).wait()
            for j in range(gh):
                hh = g * gh + j
                qv = q_ref[0, :, hh, :].astype(jnp.bfloat16)
                kv = kst[slot, :, j * d:(j + 1) * d].astype(jnp.bfloat16)
                vv = vst[slot, :, j * d:(j + 1) * d].astype(jnp.bfloat16)
                s = lax.dot_general(
                    qv, kv, (((1,), (1,)), ((), ())),
                    preferred_element_type=jnp.float32,
                ) * scale
                m = jnp.max(s, axis=1, keepdims=True)
                p = jnp.exp(s - m)
                lsum = jnp.sum(p, axis=1, keepdims=True)
                o = lax.dot_general(
                    p.astype(jnp.bfloat16), vv, (((1,), (0,)), ((), ())),
                    preferred_element_type=jnp.float32,
                )
                o_loc[bb, hh, :, :] = o.astype(jnp.bfloat16)
                ms.append(m)
                ls.append(lsum)
        st_loc[bb] = jnp.concatenate(ms + ls, axis=1)

        rdma_o(bb).start()
        rdma_st(bb).start()

        @pl.when(bb == b - 1)
        def _():
            for i in range(b):
                rdma_o(i).wait()
                rdma_st(i).wait()
            m0 = st_loc[:, :, :h]
            l0 = st_loc[:, :, h:]
            m1 = st_rx[:, :, :h]
            l1 = st_rx[:, :, h:]
            mx = jnp.maximum(m0, m1)
            e0 = jnp.exp(m0 - mx)
            e1 = jnp.exp(m1 - mx)
            denom = e0 * l0 + e1 * l1
            w0 = e0 / denom
            w1 = e1 / denom
            for hh in range(h):
                out_ref[:, :, hh, :] = (
                    o_loc[:, hh].astype(jnp.float32) * w0[:, :, hh:hh + 1]
                    + o_rx[:, hh].astype(jnp.float32) * w1[:, :, hh:hh + 1]
                )

    return pl.pallas_call(
        body,
        grid=(b,),
        in_specs=[
            pl.BlockSpec((1, q, h, d), lambda i: (i, 0, 0, 0)),
            pl.BlockSpec(memory_space=pl.ANY),
            pl.BlockSpec(memory_space=pl.ANY),
        ],
        out_specs=pl.BlockSpec((b, q, h, d), lambda i: (0, 0, 0, 0)),
        out_shape=jax.ShapeDtypeStruct((b, q, h, d), jnp.float32),
        scratch_shapes=[
            pltpu.VMEM((4, skv, gh * d), jnp.float32),
            pltpu.VMEM((4, skv, gh * d), jnp.float32),
            pltpu.SemaphoreType.DMA((4,)),
            pltpu.SemaphoreType.DMA((4,)),
            pltpu.VMEM((b, h, q, d), jnp.bfloat16),
            pltpu.VMEM((b, q, 2 * h), jnp.float32),
            pltpu.VMEM((b, h, q, d), jnp.bfloat16),
            pltpu.VMEM((b, q, 2 * h), jnp.float32),
            pltpu.SemaphoreType.DMA((b,)),
            pltpu.SemaphoreType.DMA((b,)),
            pltpu.SemaphoreType.DMA((b,)),
            pltpu.SemaphoreType.DMA((b,)),
        ],
        compiler_params=pltpu.CompilerParams(
            collective_id=0,
            vmem_limit_bytes=64 * 1024 * 1024,
        ),
    )(Q, K, V)
